# baseline (speedup 1.0000x reference)
import jax
import jax.numpy as jnp
from jax.experimental import pallas as pl
from jax.experimental.pallas import tpu as pltpu

_EPS = 1e-5
_SMOOTHING = 0.9
_TILE_F = 512
_VMEM_LIMIT = 64 * 1024 * 1024


def _cbn_kernel(x_ref, rm_ref, rv_ref, y_hbm, nrm_ref, nrv_ref,
                y_buf, out_sem):
    j = pl.program_id(0)
    tf = x_ref.shape[1]

    x = x_ref[...]
    n = jnp.float32(x.shape[0])
    s1 = jnp.sum(x, axis=0, keepdims=True)
    s2 = jnp.sum(x * x, axis=0, keepdims=True)
    mean = s1 * (1.0 / n)
    var = jnp.maximum(s2 * (1.0 / n) - mean * mean, 0.0)
    scale = jax.lax.rsqrt(var + _EPS)
    shift = -mean * scale
    y_buf[...] = x * scale + shift
    nrm_ref[...] = _SMOOTHING * rm_ref[...] + (1.0 - _SMOOTHING) * mean
    nrv_ref[...] = _SMOOTHING * rv_ref[...] + (1.0 - _SMOOTHING) * var

    copy = pltpu.make_async_copy(y_buf, y_hbm.at[:, pl.ds(j * tf, tf)],
                                 out_sem)
    copy.start()
    copy.wait()


def kernel(x, running_mean, running_var):
    n, f = x.shape
    tf = min(_TILE_F, f)
    rm = running_mean.reshape(1, f)
    rv = running_var.reshape(1, f)
    x_spec = pl.BlockSpec((n, tf), lambda j: (0, j))
    r_spec = pl.BlockSpec((1, tf), lambda j: (0, j))
    y, nrm, nrv = pl.pallas_call(
        _cbn_kernel,
        out_shape=(
            jax.ShapeDtypeStruct((n, f), x.dtype),
            jax.ShapeDtypeStruct((1, f), running_mean.dtype),
            jax.ShapeDtypeStruct((1, f), running_var.dtype),
        ),
        grid=(pl.cdiv(f, tf),),
        in_specs=[x_spec, r_spec, r_spec],
        out_specs=(pl.BlockSpec(memory_space=pl.ANY),
                   r_spec, r_spec),
        scratch_shapes=[pltpu.VMEM((n, tf), jnp.float32),
                        pltpu.SemaphoreType.DMA],
        compiler_params=pltpu.CompilerParams(
            dimension_semantics=("parallel",),
            vmem_limit_bytes=_VMEM_LIMIT),
    )(x, rm, rv)
    return y, nrm.reshape(f), nrv.reshape(f)


# speedup vs baseline: 1.0968x; 1.0968x over previous
import jax
import jax.numpy as jnp
from jax.experimental import pallas as pl
from jax.experimental.pallas import tpu as pltpu

_EPS = 1e-5
_SMOOTHING = 0.9
_TILE_F = 512
_VMEM_LIMIT = 64 * 1024 * 1024


def _cbn_kernel(x_ref, rm_ref, rv_ref, y_hbm, nrm_ref, nrv_ref,
                y_buf, out_sem):
    o = pl.program_id(0)
    i = pl.program_id(1)
    n_inner = pl.num_programs(1)
    nrows, tf = x_ref.shape
    qh = nrows // 4
    col0 = (o * n_inner + i) * tf

    def _wait(b):
        pltpu.make_async_copy(y_buf.at[b],
                              y_hbm.at[pl.ds(0, qh), pl.ds(0, tf)],
                              out_sem.at[b]).wait()

    x = x_ref[...]
    n = jnp.float32(nrows)
    s1 = jnp.sum(x, axis=0, keepdims=True)
    s2 = jnp.sum(x * x, axis=0, keepdims=True)
    mean = s1 * (1.0 / n)
    var = jnp.maximum(s2 * (1.0 / n) - mean * mean, 0.0)
    scale = jax.lax.rsqrt(var + _EPS)
    shift = -mean * scale
    nrm_ref[...] = _SMOOTHING * rm_ref[...] + (1.0 - _SMOOTHING) * mean
    nrv_ref[...] = _SMOOTHING * rv_ref[...] + (1.0 - _SMOOTHING) * var

    for k in range(4):
        b = k % 2
        if k >= 2:
            _wait(b)
        else:
            pl.when(i > 0)(lambda b=b: _wait(b))
        y_buf[b] = x[k * qh:(k + 1) * qh] * scale + shift
        pltpu.make_async_copy(y_buf.at[b],
                              y_hbm.at[pl.ds(k * qh, qh), pl.ds(col0, tf)],
                              out_sem.at[b]).start()

    @pl.when(i == n_inner - 1)
    def _():
        _wait(0)
        _wait(1)


def kernel(x, running_mean, running_var):
    n, f = x.shape
    tf = min(_TILE_F, f)
    n_blocks = pl.cdiv(f, tf)
    n_outer = 2 if n_blocks % 2 == 0 else 1
    n_inner = n_blocks // n_outer
    rm = running_mean.reshape(1, f)
    rv = running_var.reshape(1, f)
    x_spec = pl.BlockSpec((n, tf), lambda o, i: (0, o * n_inner + i))
    r_spec = pl.BlockSpec((1, tf), lambda o, i: (0, o * n_inner + i))
    y, nrm, nrv = pl.pallas_call(
        _cbn_kernel,
        out_shape=(
            jax.ShapeDtypeStruct((n, f), x.dtype),
            jax.ShapeDtypeStruct((1, f), running_mean.dtype),
            jax.ShapeDtypeStruct((1, f), running_var.dtype),
        ),
        grid=(n_outer, n_inner),
        in_specs=[x_spec, r_spec, r_spec],
        out_specs=(pl.BlockSpec(memory_space=pl.ANY),
                   r_spec, r_spec),
        scratch_shapes=[pltpu.VMEM((2, n // 4, tf), jnp.float32),
                        pltpu.SemaphoreType.DMA((2,))],
        compiler_params=pltpu.CompilerParams(
            dimension_semantics=("parallel", "arbitrary"),
            vmem_limit_bytes=_VMEM_LIMIT),
    )(x, rm, rv)
    return y, nrm.reshape(f), nrv.reshape(f)
